# Initial kernel scaffold; baseline (speedup 1.0000x reference)
import jax
import jax.numpy as jnp
from jax.experimental import pallas as pl


def kernel(mem, val, mem_labels, idx, new_labels):
    raise NotImplementedError("write your pallas kernel here")


# speedup vs baseline: 25.7264x; 25.7264x over previous
import jax
import jax.numpy as jnp
from jax import lax
from jax.experimental import pallas as pl
from jax.experimental.pallas import tpu as pltpu
from jax.experimental.pallas import tpu_sc as plsc

_M = 1000000
_D = 64
_B = 16384
_NS = 16
_NC = 2
_CHUNK = _B // _NS
_ROWS = 8
_RL = _CHUNK // _ROWS
_NV = _RL // 16
_BITS = 14
_GARBAGE = _M


def _sc_body(val_hbm, idx_hbm, nl_hbm, out_img, out_lbl,
             idx2d, ival2d, abuf, cbuf, tbuf, zbuf, lblbuf, rowbuf, T, sem):
    cid = lax.axis_index("c")
    sid = lax.axis_index("s")
    base = sid * _CHUNK

    for r in range(_ROWS):
        pltpu.sync_copy(idx_hbm.at[pl.ds(base + r * _RL, _RL)], idx2d.at[r])

    lane = lax.iota(jnp.int32, 16)
    one = jnp.full((16,), 1, jnp.int32)
    zero = jnp.full((16,), 0, jnp.int32)
    for r in range(_ROWS):
        for v in range(_NV):
            sl = pl.ds(v * 16, 16)
            ival2d[r, sl] = lane + (base + r * _RL + v * 16)
            abuf[r, sl] = one
            zbuf[r, sl] = zero

    def round_body(t, carry):
        b = (_BITS - 1) - t
        cps = [pltpu.async_copy(zbuf.at[r], T.at[idx2d.at[r]], sem)
               for r in range(_ROWS)]
        for c in cps:
            c.wait()
        plsc.subcore_barrier()
        for r in range(_ROWS):
            for v in range(_NV):
                sl = pl.ds(v * 16, 16)
                bit = lax.shift_right_logical(ival2d[r, sl],
                                              jnp.broadcast_to(b, (16,))) & one
                cbuf[r, sl] = abuf[r, sl] * bit
        cps = [pltpu.async_copy(cbuf.at[r], T.at[idx2d.at[r]], sem, add=True)
               for r in range(_ROWS)]
        for c in cps:
            c.wait()
        plsc.subcore_barrier()
        cps = [pltpu.async_copy(T.at[idx2d.at[r]], tbuf.at[r], sem)
               for r in range(_ROWS)]
        for c in cps:
            c.wait()
        for r in range(_ROWS):
            for v in range(_NV):
                sl = pl.ds(v * 16, 16)
                bit = lax.shift_right_logical(ival2d[r, sl],
                                              jnp.broadcast_to(b, (16,))) & one
                tpos = jnp.minimum(tbuf[r, sl], one)
                keep = jnp.maximum(bit, one - tpos)
                abuf[r, sl] = abuf[r, sl] * keep
        plsc.subcore_barrier()
        return carry

    lax.fori_loop(0, _BITS, round_body, jnp.int32(0))

    cps = [pltpu.async_copy(zbuf.at[r], T.at[idx2d.at[r]], sem)
           for r in range(_ROWS)]
    for c in cps:
        c.wait()
    plsc.subcore_barrier()
    for r in range(_ROWS):
        for v in range(_NV):
            sl = pl.ds(v * 16, 16)
            cbuf[r, sl] = abuf[r, sl] * ival2d[r, sl]
    cps = [pltpu.async_copy(cbuf.at[r], T.at[idx2d.at[r]], sem, add=True)
           for r in range(_ROWS)]
    for c in cps:
        c.wait()
    plsc.subcore_barrier()
    cps = [pltpu.async_copy(T.at[idx2d.at[r]], tbuf.at[r], sem)
           for r in range(_ROWS)]
    for c in cps:
        c.wait()

    for h in range(_ROWS // _NC):
        r = cid * (_ROWS // _NC) + h
        off = base + r * _RL
        pltpu.async_copy(nl_hbm.at[tbuf.at[r]], lblbuf, sem).wait()
        pltpu.sync_copy(lblbuf, out_lbl.at[pl.ds(off, _RL)])
        pltpu.async_copy(val_hbm.at[tbuf.at[r]], rowbuf, sem).wait()
        pltpu.sync_copy(rowbuf, out_img.at[pl.ds(off, _RL), :])


def kernel(mem, val, mem_labels, idx, new_labels):
    del mem, mem_labels
    f = pl.kernel(
        _sc_body,
        out_type=(jax.ShapeDtypeStruct((_B, 128), jnp.float32),
                  jax.ShapeDtypeStruct((_B,), jnp.int32)),
        mesh=plsc.VectorSubcoreMesh(core_axis_name="c", subcore_axis_name="s"),
        scratch_types=[
            pltpu.VMEM((_ROWS, _RL), jnp.int32),
            pltpu.VMEM((_ROWS, _RL), jnp.int32),
            pltpu.VMEM((_ROWS, _RL), jnp.int32),
            pltpu.VMEM((_ROWS, _RL), jnp.int32),
            pltpu.VMEM((_ROWS, _RL), jnp.int32),
            pltpu.VMEM((_ROWS, _RL), jnp.int32),
            pltpu.VMEM((_RL,), jnp.int32),
            pltpu.VMEM((_RL, 128), jnp.float32),
            pltpu.VMEM_SHARED((_M + 16,), jnp.int32),
            pltpu.SemaphoreType.DMA,
        ],
    )
    val_p = jnp.pad(val, ((0, 0), (0, 128 - _D)))
    ret_imgs_p, ret_labels = f(val_p, idx, new_labels)
    return (ret_imgs_p[:, :_D], ret_labels)
